# Initial kernel scaffold; baseline (speedup 1.0000x reference)
import jax
import jax.numpy as jnp
from jax.experimental import pallas as pl


def kernel(h, edge_index, snorm_n, W_self, W_func, W_att):
    raise NotImplementedError("write your pallas kernel here")


# speedup vs baseline: 15.2441x; 15.2441x over previous
import jax
import jax.numpy as jnp
from jax import lax
from jax.experimental import pallas as pl
from jax.experimental.pallas import tpu as pltpu
from jax.experimental.pallas import tpu_sc as plsc

N = 10000
E = 320000
D = 128
NC, NS = 2, 16
NW = NC * NS
RPAD = 2560
EPAD = RPAD * 128 - E
SUP = 8
NSUPW = RPAD // SUP // NW
NPAD = 10240
BLK = 2000
GRID = N // BLK


def _mm_body(h_ref, ws_ref, wf_ref, a2_ref, hs_ref, z_ref, uv_ref):
    hb = h_ref[...]
    ws = ws_ref[...]
    wf = wf_ref[...]
    a2 = a2_ref[...]
    hs_ref[...] = lax.dot_general(hb, ws, (((1,), (1,)), ((), ())),
                                  preferred_element_type=jnp.float32)
    z_ref[...] = lax.dot_general(hb, wf, (((1,), (1,)), ((), ())),
                                 preferred_element_type=jnp.float32)
    p2 = lax.dot_general(a2, wf, (((1,), (0,)), ((), ())),
                         preferred_element_type=jnp.float32)
    uv_ref[...] = lax.dot_general(hb, p2, (((1,), (1,)), ((), ())),
                                  preferred_element_type=jnp.float32)


def _allreduce(vec, buf_v, iota16, op):
    for sft in (1, 2, 4, 8):
        buf_v[...] = vec
        vec = op(vec, plsc.load_gather(buf_v, [iota16 ^ sft]))
    return vec


def _edge_body(uvf_hbm, ei2_hbm, ex_hbm, den0_hbm, den1_hbm, sump_hbm,
               uvf_v, srcs_v, dsts_v, ex2_v, zero_v, sacc_v, den_sh, sem):
    cid = lax.axis_index("c")
    sid = lax.axis_index("s")
    wid = sid * NC + cid
    pltpu.sync_copy(uvf_hbm, uvf_v)

    def zb(i, c):
        zero_v[pl.ds(i * 16, 16)] = jnp.zeros((16,), jnp.float32)
        return c
    lax.fori_loop(0, 40, zb, 0)
    pltpu.sync_copy(zero_v, den_sh.at[pl.ds(sid * 640, 640)])

    def mb(i, m):
        return jnp.maximum(m, uvf_v[pl.ds(i * 16, 16)])
    mx16 = lax.fori_loop(0, 2 * N // 16, mb,
                         jnp.full((16,), -1e30, jnp.float32))
    iota16 = lax.iota(jnp.int32, 16)
    mm = 2.0 * _allreduce(mx16, sacc_v, iota16, jnp.maximum)
    m_shift = jnp.where(mm > 0, mm, mm * 0.01)
    plsc.subcore_barrier()

    def sup_body(t, ssum):
        sc = wid * NSUPW + t
        r0 = sc * SUP
        pltpu.sync_copy(ei2_hbm.at[0, pl.ds(r0, SUP)], srcs_v)
        pltpu.sync_copy(ei2_hbm.at[1, pl.ds(r0, SUP)], dsts_v)

        def gb(g, acc):
            j = g // 8
            c = g % 8
            s16 = srcs_v[j, pl.ds(c * 16, 16)]
            d16 = dsts_v[j, pl.ds(c * 16, 16)]
            uu = plsc.load_gather(uvf_v, [s16 * 2])
            vv = plsc.load_gather(uvf_v, [d16 * 2 + 1])
            e16 = uu + vv
            e16 = jnp.where(e16 > 0, e16, e16 * 0.01)
            x16 = jnp.exp(e16 - m_shift)
            base = (r0 + j) * 128 + c * 16
            x16 = jnp.where(base + iota16 < E, x16, 0.0)
            ex2_v[j, pl.ds(c * 16, 16)] = x16
            return acc + x16
        ssum = lax.fori_loop(0, SUP * 8, gb, ssum)
        pltpu.sync_copy(ex2_v, ex_hbm.at[pl.ds(r0, SUP)])
        descs = [pltpu.async_copy(ex2_v.at[j], den_sh.at[dsts_v.at[j]],
                                  sem, add=True) for j in range(SUP)]
        for dsc in descs:
            dsc.wait()
        return ssum
    ssum = lax.fori_loop(0, NSUPW, sup_body, jnp.zeros((16,), jnp.float32))
    plsc.subcore_barrier()

    @pl.when(cid == 0)
    def _():
        pltpu.sync_copy(den_sh.at[pl.ds(sid * 640, 640)],
                        den0_hbm.at[pl.ds(sid * 640, 640)])

    @pl.when(cid == 1)
    def _():
        pltpu.sync_copy(den_sh.at[pl.ds(sid * 640, 640)],
                        den1_hbm.at[pl.ds(sid * 640, 640)])
    sacc_v[...] = ssum
    pltpu.sync_copy(sacc_v, sump_hbm.at[pl.ds(wid * 16, 16)])


def _agg_body(z_hbm, ei2_hbm, ex_hbm, den0_hbm, den1_hbm, sump_hbm,
              aggp_hbm, ae_hbm,
              den_v, tmp_v, srcs_v, dsts_v, exs_v, ak2_v, wk_v, rows_v,
              zr_v, ssm_v, agg_sh, sem):
    cid = lax.axis_index("c")
    sid = lax.axis_index("s")
    wid = sid * NC + cid
    pltpu.sync_copy(sump_hbm, ssm_v)

    def sb(i, a):
        return a + ssm_v[pl.ds(i * 16, 16)]
    acc = lax.fori_loop(0, NW, sb, jnp.zeros((16,), jnp.float32))
    iota16 = lax.iota(jnp.int32, 16)
    rcp_s = 1.0 / _allreduce(acc, wk_v.at[pl.ds(0, 16)], iota16, jnp.add)

    pltpu.sync_copy(den0_hbm, den_v)
    pltpu.sync_copy(den1_hbm, tmp_v)

    def db(i, c):
        den_v[pl.ds(i * 16, 16)] = (den_v[pl.ds(i * 16, 16)]
                                    + tmp_v[pl.ds(i * 16, 16)])
        return c
    lax.fori_loop(0, NPAD // 16, db, 0)

    def zb(i, c):
        zr_v[i // 8, pl.ds((i % 8) * 16, 16)] = jnp.zeros((16,), jnp.float32)
        return c
    lax.fori_loop(0, 256, zb, 0)

    def az(i, c):
        pltpu.sync_copy(zr_v, agg_sh.at[pl.ds(sid * 640 + i * 32, 32)])
        return c
    lax.fori_loop(0, 20, az, 0)
    plsc.subcore_barrier()

    def sup_body(t, carry):
        r0 = (wid * NSUPW + t) * SUP
        pltpu.sync_copy(ei2_hbm.at[0, pl.ds(r0, SUP)], srcs_v)
        pltpu.sync_copy(ei2_hbm.at[1, pl.ds(r0, SUP)], dsts_v)
        pltpu.sync_copy(ex_hbm.at[pl.ds(r0, SUP)], exs_v)
        for j in range(SUP):
            g = pltpu.async_copy(z_hbm.at[srcs_v.at[j]], rows_v, sem)

            def wb(i, cc):
                d16 = dsts_v[j, pl.ds(i * 16, 16)]
                den16 = plsc.load_gather(den_v, [d16])
                x16 = exs_v[j, pl.ds(i * 16, 16)]
                wk_v[pl.ds(i * 16, 16)] = x16 / den16
                ak2_v[j, pl.ds(i * 16, 16)] = x16 * rcp_s
                return cc
            lax.fori_loop(0, 8, wb, 0)
            g.wait()

            def rb(i, cc):
                wb16 = plsc.load_gather(wk_v, [jnp.full((16,), i, jnp.int32)])
                for col in range(8):
                    rows_v[i, pl.ds(col * 16, 16)] = (
                        rows_v[i, pl.ds(col * 16, 16)] * wb16)
                return cc
            lax.fori_loop(0, 128, rb, 0)
            pltpu.sync_copy(rows_v, agg_sh.at[dsts_v.at[j]], add=True)
        pltpu.sync_copy(ak2_v, ae_hbm.at[pl.ds(r0, SUP)])
        return carry
    lax.fori_loop(0, NSUPW, sup_body, 0)
    plsc.subcore_barrier()
    pltpu.sync_copy(agg_sh.at[pl.ds(sid * 640, 640)],
                    aggp_hbm.at[cid, pl.ds(sid * 640, 640)])


def _fin_body(h_ref, hs_ref, den_ref, aggp_ref, out_ref):
    den = den_ref[...]
    hn = hs_ref[...] + aggp_ref[0] + aggp_ref[1]
    hn = jnp.where(den > 0, hn, h_ref[...])
    out_ref[...] = h_ref[...] + jnp.maximum(hn, 0.0)


def kernel(h, edge_index, snorm_n, W_self, W_func, W_att):
    a2 = W_att.reshape(2, D)
    pad = jnp.tile(jnp.arange(EPAD, dtype=jnp.int32)[None, :] % N, (2, 1))
    ei2 = jnp.concatenate([edge_index, pad], axis=1).reshape(2, RPAD, 128)

    hs, z, uv = pl.pallas_call(
        _mm_body,
        grid=(GRID,),
        in_specs=[
            pl.BlockSpec((BLK, D), lambda i: (i, 0)),
            pl.BlockSpec((D, D), lambda i: (0, 0)),
            pl.BlockSpec((D, D), lambda i: (0, 0)),
            pl.BlockSpec((2, D), lambda i: (0, 0)),
        ],
        out_specs=[
            pl.BlockSpec((BLK, D), lambda i: (i, 0)),
            pl.BlockSpec((BLK, D), lambda i: (i, 0)),
            pl.BlockSpec((BLK, 2), lambda i: (i, 0)),
        ],
        out_shape=[
            jax.ShapeDtypeStruct((N, D), jnp.float32),
            jax.ShapeDtypeStruct((N, D), jnp.float32),
            jax.ShapeDtypeStruct((N, 2), jnp.float32),
        ],
    )(h, W_self, W_func, a2)
    uvf = uv.reshape(2 * N)

    mesh = plsc.VectorSubcoreMesh(core_axis_name="c", subcore_axis_name="s")
    ex, den0, den1, sump = pl.kernel(
        _edge_body,
        out_type=[
            jax.ShapeDtypeStruct((RPAD, 128), jnp.float32),
            jax.ShapeDtypeStruct((NPAD,), jnp.float32),
            jax.ShapeDtypeStruct((NPAD,), jnp.float32),
            jax.ShapeDtypeStruct((NW * 16,), jnp.float32),
        ],
        mesh=mesh,
        compiler_params=pltpu.CompilerParams(needs_layout_passes=False),
        scratch_types=[
            pltpu.VMEM((2 * N,), jnp.float32),
            pltpu.VMEM((SUP, 128), jnp.int32),
            pltpu.VMEM((SUP, 128), jnp.int32),
            pltpu.VMEM((SUP, 128), jnp.float32),
            pltpu.VMEM((640,), jnp.float32),
            pltpu.VMEM((16,), jnp.float32),
            pltpu.VMEM_SHARED((NPAD,), jnp.float32),
            pltpu.SemaphoreType.DMA,
        ],
    )(uvf, ei2)

    aggp, ae = pl.kernel(
        _agg_body,
        out_type=[
            jax.ShapeDtypeStruct((2, NPAD, D), jnp.float32),
            jax.ShapeDtypeStruct((RPAD, 128), jnp.float32),
        ],
        mesh=mesh,
        compiler_params=pltpu.CompilerParams(needs_layout_passes=False),
        scratch_types=[
            pltpu.VMEM((NPAD,), jnp.float32),
            pltpu.VMEM((NPAD,), jnp.float32),
            pltpu.VMEM((SUP, 128), jnp.int32),
            pltpu.VMEM((SUP, 128), jnp.int32),
            pltpu.VMEM((SUP, 128), jnp.float32),
            pltpu.VMEM((SUP, 128), jnp.float32),
            pltpu.VMEM((128,), jnp.float32),
            pltpu.VMEM((128, D), jnp.float32),
            pltpu.VMEM((32, D), jnp.float32),
            pltpu.VMEM((NW * 16,), jnp.float32),
            pltpu.VMEM_SHARED((NPAD, D), jnp.float32),
            pltpu.SemaphoreType.DMA,
        ],
    )(z, ei2, ex, den0, den1, sump)

    den2d = (den0[:N] + den1[:N]).reshape(N, 1)
    h_out = pl.pallas_call(
        _fin_body,
        grid=(GRID,),
        in_specs=[
            pl.BlockSpec((BLK, D), lambda i: (i, 0)),
            pl.BlockSpec((BLK, D), lambda i: (i, 0)),
            pl.BlockSpec((BLK, 1), lambda i: (i, 0)),
            pl.BlockSpec((2, BLK, D), lambda i: (0, i, 0)),
        ],
        out_specs=pl.BlockSpec((BLK, D), lambda i: (i, 0)),
        out_shape=jax.ShapeDtypeStruct((N, D), jnp.float32),
    )(h, hs, den2d, aggp)

    a_edge = ae.reshape(RPAD * 128)[:E].reshape(E, 1)
    return (h_out, a_edge)
